# Initial kernel scaffold; baseline (speedup 1.0000x reference)
import jax
import jax.numpy as jnp
from jax.experimental import pallas as pl


def kernel(x, edge_index, W1, b1, W2, b2, W3, b3):
    raise NotImplementedError("write your pallas kernel here")


# speedup vs baseline: 24.8112x; 24.8112x over previous
import functools

import jax
import jax.numpy as jnp
from jax import lax
from jax.experimental import pallas as pl
from jax.experimental.pallas import tpu as pltpu
from jax.experimental.pallas import tpu_sc as plsc

N = 10000
P = 10240
E = 320000
D = 128
DH = 64
NC, NS = 2, 16
K = 125
NB = E // (NS * K)
NBD = E // (NC * NS * K)
RPT = P // NS

_mesh = plsc.VectorSubcoreMesh(core_axis_name="c", subcore_axis_name="s")

_f32 = jnp.float32


def _zero_fill(buf, n_rows, width):
    def body(i, _):
        for j in range(width // 16):
            buf[i, pl.ds(j * 16, 16)] = jnp.zeros((16,), _f32)
        return 0
    lax.fori_loop(0, n_rows, body, 0, unroll=2)


@functools.partial(
    pl.kernel,
    out_type=jax.ShapeDtypeStruct((NC, P), _f32),
    mesh=_mesh,
    scratch_types=[
        pltpu.VMEM((NBD, K), jnp.int32),
        pltpu.VMEM((1, 128), _f32),
        pltpu.VMEM((128, 64), _f32),
        pltpu.VMEM_SHARED((P,), _f32),
    ],
)
def _deg_kernel(dst_hbm, out_hbm, idx_v, ones_v, zer_v, acc_sh):
    c = lax.axis_index("c")
    s = lax.axis_index("s")
    pltpu.sync_copy(dst_hbm.at[c, s], idx_v)
    for j in range(8):
        ones_v[0, pl.ds(j * 16, 16)] = jnp.full((16,), 1.0, _f32)
    _zero_fill(zer_v, 10, 64)
    for k in range(10):
        pltpu.sync_copy(zer_v.at[0], acc_sh.at[pl.ds(s * RPT + k * 64, 64)])
    plsc.subcore_barrier()

    def body(j, _):
        pltpu.sync_copy(ones_v.at[0, pl.ds(0, K)], acc_sh.at[idx_v.at[j]],
                        add=True)
        return 0
    lax.fori_loop(0, NBD, body, 0)
    plsc.subcore_barrier()
    pltpu.sync_copy(acc_sh.at[pl.ds(s * RPT, RPT)],
                    out_hbm.at[c, pl.ds(s * RPT, RPT)])


def _make_edge_kernel(with_s):
    out_type = [jax.ShapeDtypeStruct((NC, P, DH), _f32)]
    scratch = [
        pltpu.VMEM((NB, K), jnp.int32),
        pltpu.VMEM((NB, K), jnp.int32),
        pltpu.VMEM((2, K, DH), _f32),
        pltpu.VMEM((128, 64), _f32),
        pltpu.VMEM_SHARED((P, DH), _f32),
        pltpu.SemaphoreType.DMA((2,)),
    ]
    if with_s:
        out_type.append(jax.ShapeDtypeStruct((NC, P), _f32))
        scratch += [
            pltpu.VMEM((2, K), _f32),
            pltpu.VMEM_SHARED((P,), _f32),
            pltpu.SemaphoreType.DMA((2,)),
        ]

    def body(g_hbm, src_hbm, dst_hbm, dinv_hbm, *refs):
        if with_s:
            (r_out, s_out, src_v, dst_v, rows_v, zer_v, acc_sh, gsem,
             dval_v, s_sh, dsem) = refs
        else:
            r_out, src_v, dst_v, rows_v, zer_v, acc_sh, gsem = refs
        c = lax.axis_index("c")
        s = lax.axis_index("s")
        pltpu.sync_copy(src_hbm.at[s], src_v)
        pltpu.sync_copy(dst_hbm.at[s], dst_v)
        _zero_fill(zer_v, 128, 64)
        for k in range(5):
            pltpu.sync_copy(zer_v, acc_sh.at[pl.ds(s * RPT + k * 128, 128)])
        if with_s:
            for k in range(10):
                pltpu.sync_copy(zer_v.at[0],
                                s_sh.at[pl.ds(s * RPT + k * 64, 64)])
        plsc.subcore_barrier()

        def loop(i, _):
            for p in range(2):
                j = i * 2 + p
                pltpu.async_copy(g_hbm.at[c].at[src_v.at[j]], rows_v.at[p],
                                 gsem.at[p])
            if with_s:
                @pl.when(i < NBD // 2)
                def _():
                    for p in range(2):
                        j = c * NBD + i * 2 + p
                        pltpu.async_copy(dinv_hbm.at[dst_v.at[j]],
                                         dval_v.at[p], dsem.at[p])
            for p in range(2):
                j = i * 2 + p
                pltpu.make_async_copy(g_hbm.at[c].at[src_v.at[j]],
                                      rows_v.at[p], gsem.at[p]).wait()
                pltpu.sync_copy(rows_v.at[p], acc_sh.at[dst_v.at[j]],
                                add=True)
            if with_s:
                @pl.when(i < NBD // 2)
                def _():
                    for p in range(2):
                        j = c * NBD + i * 2 + p
                        pltpu.make_async_copy(dinv_hbm.at[dst_v.at[j]],
                                              dval_v.at[p], dsem.at[p]).wait()
                        pltpu.sync_copy(dval_v.at[p], s_sh.at[src_v.at[j]],
                                        add=True)
            return 0
        lax.fori_loop(0, NB // 2, loop, 0)
        plsc.subcore_barrier()
        for k in range(5):
            pltpu.sync_copy(acc_sh.at[pl.ds(s * RPT + k * 128, 128)],
                            r_out.at[c, pl.ds(s * RPT + k * 128, 128)])
        if with_s:
            pltpu.sync_copy(s_sh.at[pl.ds(s * RPT, RPT)],
                            s_out.at[c, pl.ds(s * RPT, RPT)])

    return pl.kernel(body, out_type=out_type, mesh=_mesh,
                     scratch_types=scratch,
                     compiler_params=pltpu.CompilerParams(
                         use_tc_tiling_on_sc=False))


_edge_kernel_s = _make_edge_kernel(True)
_edge_kernel = _make_edge_kernel(False)


BR = 640
GRID = P // BR


def _split(t, dinv, out_ref):
    out_ref[0] = t[:, :DH] * dinv
    out_ref[1] = t[:, DH:] * dinv


def _tc1_body(x_ref, w1_ref, ip_ref, g1_ref, dinv_ref):
    ip = ip_ref[...]
    deg = 1.0 + ip[0] + ip[1]
    dinv = lax.rsqrt(deg)
    dinv_ref[...] = dinv
    t = jnp.dot(x_ref[...], w1_ref[...], preferred_element_type=_f32,
                precision=lax.Precision.HIGHEST)
    _split(t, dinv, g1_ref)


_tc1 = pl.pallas_call(
    _tc1_body,
    grid=(GRID,),
    in_specs=[
        pl.BlockSpec((BR, D), lambda i: (i, 0)),
        pl.BlockSpec((D, D), lambda i: (0, 0)),
        pl.BlockSpec((2, BR, 1), lambda i: (0, i, 0)),
    ],
    out_specs=[
        pl.BlockSpec((2, BR, DH), lambda i: (0, i, 0)),
        pl.BlockSpec((BR, 1), lambda i: (i, 0)),
    ],
    out_shape=[
        jax.ShapeDtypeStruct((NC, P, DH), _f32),
        jax.ShapeDtypeStruct((P, 1), _f32),
    ],
)


def _tc2_body(rp_ref, g1_ref, dinv_ref, b1_ref, w2_ref, g2_ref):
    rp = rp_ref[...]
    g1 = g1_ref[...]
    dinv = dinv_ref[...]
    r = jnp.concatenate([rp[0] + g1[0], rp[1] + g1[1]], axis=1)
    h1 = jax.nn.relu(dinv * r + b1_ref[...])
    t = jnp.dot(h1, w2_ref[...], preferred_element_type=_f32,
                precision=lax.Precision.HIGHEST)
    _split(t, dinv, g2_ref)


_tc2 = pl.pallas_call(
    _tc2_body,
    grid=(GRID,),
    in_specs=[
        pl.BlockSpec((2, BR, DH), lambda i: (0, i, 0)),
        pl.BlockSpec((2, BR, DH), lambda i: (0, i, 0)),
        pl.BlockSpec((BR, 1), lambda i: (i, 0)),
        pl.BlockSpec((1, D), lambda i: (0, 0)),
        pl.BlockSpec((D, D), lambda i: (0, 0)),
    ],
    out_specs=pl.BlockSpec((2, BR, DH), lambda i: (0, i, 0)),
    out_shape=jax.ShapeDtypeStruct((NC, P, DH), _f32),
)


def _tc3_body(rp_ref, g2_ref, dinv_ref, b2_ref, sp_ref, w3_ref, b3_ref,
              out_ref, acc_ref):
    i = pl.program_id(0)
    rp = rp_ref[...]
    g2 = g2_ref[...]
    dinv = dinv_ref[...]
    r = jnp.concatenate([rp[0] + g2[0], rp[1] + g2[1]], axis=1)
    h2 = jax.nn.relu(dinv * r + b2_ref[...])
    sp = sp_ref[...]
    w = dinv * (sp[0] + sp[1] + dinv)
    rows = i * BR + lax.broadcasted_iota(jnp.int32, (BR, 1), 0)
    w = jnp.where(rows < N, w, 0.0)
    contrib = jnp.sum(w * h2, axis=0, keepdims=True)

    @pl.when(i == 0)
    def _():
        acc_ref[...] = contrib

    @pl.when(i > 0)
    def _():
        acc_ref[...] = acc_ref[...] + contrib

    @pl.when(i == GRID - 1)
    def _():
        u = acc_ref[...] * (1.0 / N)
        out_ref[...] = jnp.dot(u, w3_ref[...], preferred_element_type=_f32,
                               precision=lax.Precision.HIGHEST) + b3_ref[...]


_tc3 = pl.pallas_call(
    _tc3_body,
    grid=(GRID,),
    in_specs=[
        pl.BlockSpec((2, BR, DH), lambda i: (0, i, 0)),
        pl.BlockSpec((2, BR, DH), lambda i: (0, i, 0)),
        pl.BlockSpec((BR, 1), lambda i: (i, 0)),
        pl.BlockSpec((1, D), lambda i: (0, 0)),
        pl.BlockSpec((2, BR, 1), lambda i: (0, i, 0)),
        pl.BlockSpec((D, 64), lambda i: (0, 0)),
        pl.BlockSpec((1, 64), lambda i: (0, 0)),
    ],
    out_specs=pl.BlockSpec((1, 64), lambda i: (0, 0)),
    out_shape=jax.ShapeDtypeStruct((1, 64), _f32),
    scratch_shapes=[pltpu.VMEM((1, D), _f32)],
)


def kernel(x, edge_index, W1, b1, W2, b2, W3, b3):
    src = edge_index[0].astype(jnp.int32).reshape(NS, NB, K)
    dst = edge_index[1].astype(jnp.int32).reshape(NS, NB, K)
    dst_deg = edge_index[1].astype(jnp.int32).reshape(NC, NS, NBD, K)
    xp = jnp.pad(x, ((0, P - N), (0, 0)))

    indeg_p = _deg_kernel(dst_deg)
    g1, dinv = _tc1(xp, W1, indeg_p.reshape(NC, P, 1))
    dinv_flat = dinv.reshape(P)
    r1, s_raw = _edge_kernel_s(g1, src, dst, dinv_flat)
    g2 = _tc2(r1, g1, dinv, b1.reshape(1, D), W2)
    (r2,) = _edge_kernel(g2, src, dst, dinv_flat)
    out = _tc3(r2, g2, dinv, b2.reshape(1, D), s_raw.reshape(NC, P, 1),
               W3, b3.reshape(1, 64))
    return out


# speedup vs baseline: 30.2828x; 1.2205x over previous
import functools

import jax
import jax.numpy as jnp
from jax import lax
from jax.experimental import pallas as pl
from jax.experimental.pallas import tpu as pltpu
from jax.experimental.pallas import tpu_sc as plsc

N = 10000
P = 10240
E = 320000
D = 128
DH = 64
NC, NS = 2, 16
K = 125
NB = E // (NS * K)
NBD = E // (NC * NS * K)
RPT = P // NS

_mesh = plsc.VectorSubcoreMesh(core_axis_name="c", subcore_axis_name="s")

_f32 = jnp.float32


def _zero_fill(buf, n_rows, width):
    def body(i, _):
        for j in range(width // 16):
            buf[i, pl.ds(j * 16, 16)] = jnp.zeros((16,), _f32)
        return 0
    lax.fori_loop(0, n_rows, body, 0, unroll=2)


@functools.partial(
    pl.kernel,
    out_type=jax.ShapeDtypeStruct((NC, P), _f32),
    mesh=_mesh,
    scratch_types=[
        pltpu.VMEM((NBD, K), jnp.int32),
        pltpu.VMEM((1, 128), _f32),
        pltpu.VMEM((128, 64), _f32),
        pltpu.VMEM_SHARED((P,), _f32),
        pltpu.SemaphoreType.DMA,
    ],
)
def _deg_kernel(dst_hbm, out_hbm, idx_v, ones_v, zer_v, acc_sh, sem):
    c = lax.axis_index("c")
    s = lax.axis_index("s")
    pltpu.sync_copy(dst_hbm.at[c, s], idx_v)
    for j in range(8):
        ones_v[0, pl.ds(j * 16, 16)] = jnp.full((16,), 1.0, _f32)
    _zero_fill(zer_v, 10, 64)
    for k in range(10):
        pltpu.sync_copy(zer_v.at[0], acc_sh.at[pl.ds(s * RPT + k * 64, 64)])
    plsc.subcore_barrier()

    def body(j, _):
        pltpu.async_copy(ones_v.at[0, pl.ds(0, K)], acc_sh.at[idx_v.at[j]],
                         sem, add=True)
        return 0
    lax.fori_loop(0, NBD, body, 0)

    def drain(j, _):
        pltpu.make_async_copy(ones_v.at[0, pl.ds(0, K)],
                              acc_sh.at[idx_v.at[0]], sem).wait()
        return 0
    lax.fori_loop(0, NBD, drain, 0)
    plsc.subcore_barrier()
    pltpu.sync_copy(acc_sh.at[pl.ds(s * RPT, RPT)],
                    out_hbm.at[c, pl.ds(s * RPT, RPT)])


NBUF = 4
NROUNDS = NB // NBUF
NBS = NBD


def _make_edge_kernel(with_s):
    out_type = [jax.ShapeDtypeStruct((NC, P, DH), _f32)]
    scratch = [
        pltpu.VMEM((NB, K), jnp.int32),
        pltpu.VMEM((NB, K), jnp.int32),
        pltpu.VMEM((NBUF, K, DH), _f32),
        pltpu.VMEM((64, 64), _f32),
        pltpu.VMEM_SHARED((P, DH), _f32),
        pltpu.SemaphoreType.DMA((NBUF,)),
        pltpu.SemaphoreType.DMA((NBUF,)),
    ]
    if with_s:
        out_type.append(jax.ShapeDtypeStruct((NC, P), _f32))
        scratch += [
            pltpu.VMEM((NBS, K), _f32),
            pltpu.VMEM_SHARED((P,), _f32),
            pltpu.SemaphoreType.DMA,
            pltpu.SemaphoreType.DMA,
        ]

    def body(g_hbm, src_hbm, dst_hbm, dinv_hbm, *refs):
        if with_s:
            (r_out, s_out, src_v, dst_v, rows_v, zer_v, acc_sh, gsem, ssem,
             vals_v, s_sh, sgsem, sssem) = refs
        else:
            (r_out, src_v, dst_v, rows_v, zer_v, acc_sh, gsem,
             ssem) = refs
        c = lax.axis_index("c")
        s = lax.axis_index("s")
        pltpu.sync_copy(src_hbm.at[s], src_v)
        pltpu.sync_copy(dst_hbm.at[s], dst_v)

        if with_s:
            def sgather(j, _):
                pltpu.async_copy(dinv_hbm.at[dst_v.at[c * NBS + j]],
                                 vals_v.at[j], sgsem)
                return 0
            lax.fori_loop(0, NBS, sgather, 0)

        _zero_fill(zer_v, 64, 64)
        for k in range(10):
            pltpu.sync_copy(zer_v, acc_sh.at[pl.ds(s * RPT + k * 64, 64)])
        if with_s:
            for k in range(10):
                pltpu.sync_copy(zer_v.at[0],
                                s_sh.at[pl.ds(s * RPT + k * 64, 64)])
        plsc.subcore_barrier()

        for u in range(NBUF):
            pltpu.async_copy(g_hbm.at[c].at[src_v.at[u]], rows_v.at[u],
                             gsem.at[u])

        def loop(i, _):
            for u in range(NBUF):
                j = i * NBUF + u
                pltpu.make_async_copy(g_hbm.at[c].at[src_v.at[j]],
                                      rows_v.at[u], gsem.at[u]).wait()
                pltpu.async_copy(rows_v.at[u], acc_sh.at[dst_v.at[j]],
                                 ssem.at[u], add=True)
            for u in range(NBUF):
                j = i * NBUF + u
                pltpu.make_async_copy(rows_v.at[u], acc_sh.at[dst_v.at[j]],
                                      ssem.at[u]).wait()

                @pl.when(i < NROUNDS - 1)
                def _():
                    jn = (i + 1) * NBUF + u
                    pltpu.async_copy(g_hbm.at[c].at[src_v.at[jn]],
                                     rows_v.at[u], gsem.at[u])
            return 0
        lax.fori_loop(0, NROUNDS, loop, 0)

        if with_s:
            def sdrain_g(j, _):
                pltpu.make_async_copy(dinv_hbm.at[dst_v.at[c * NBS]],
                                      vals_v.at[0], sgsem).wait()
                return 0
            lax.fori_loop(0, NBS, sdrain_g, 0)

            def sfire(j, _):
                pltpu.async_copy(vals_v.at[j], s_sh.at[src_v.at[c * NBS + j]],
                                 sssem, add=True)
                return 0
            lax.fori_loop(0, NBS, sfire, 0)

            def sdrain_s(j, _):
                pltpu.make_async_copy(vals_v.at[0], s_sh.at[src_v.at[c * NBS]],
                                      sssem).wait()
                return 0
            lax.fori_loop(0, NBS, sdrain_s, 0)
        plsc.subcore_barrier()
        for k in range(5):
            pltpu.sync_copy(acc_sh.at[pl.ds(s * RPT + k * 128, 128)],
                            r_out.at[c, pl.ds(s * RPT + k * 128, 128)])
        if with_s:
            pltpu.sync_copy(s_sh.at[pl.ds(s * RPT, RPT)],
                            s_out.at[c, pl.ds(s * RPT, RPT)])

    return pl.kernel(body, out_type=out_type, mesh=_mesh,
                     scratch_types=scratch,
                     compiler_params=pltpu.CompilerParams(
                         use_tc_tiling_on_sc=False))


_edge_kernel_s = _make_edge_kernel(True)
_edge_kernel = _make_edge_kernel(False)


BR = 640
GRID = P // BR


def _split(t, dinv, out_ref):
    out_ref[0] = t[:, :DH] * dinv
    out_ref[1] = t[:, DH:] * dinv


def _tc1_body(x_ref, w1_ref, ip_ref, g1_ref, dinv_ref):
    ip = ip_ref[...]
    deg = 1.0 + ip[0] + ip[1]
    dinv = lax.rsqrt(deg)
    dinv_ref[...] = dinv
    t = jnp.dot(x_ref[...], w1_ref[...], preferred_element_type=_f32,
                precision=lax.Precision.HIGHEST)
    _split(t, dinv, g1_ref)


_tc1 = pl.pallas_call(
    _tc1_body,
    grid=(GRID,),
    in_specs=[
        pl.BlockSpec((BR, D), lambda i: (i, 0)),
        pl.BlockSpec((D, D), lambda i: (0, 0)),
        pl.BlockSpec((2, BR, 1), lambda i: (0, i, 0)),
    ],
    out_specs=[
        pl.BlockSpec((2, BR, DH), lambda i: (0, i, 0)),
        pl.BlockSpec((BR, 1), lambda i: (i, 0)),
    ],
    out_shape=[
        jax.ShapeDtypeStruct((NC, P, DH), _f32),
        jax.ShapeDtypeStruct((P, 1), _f32),
    ],
)


def _tc2_body(rp_ref, g1_ref, dinv_ref, b1_ref, w2_ref, g2_ref):
    rp = rp_ref[...]
    g1 = g1_ref[...]
    dinv = dinv_ref[...]
    r = jnp.concatenate([rp[0] + g1[0], rp[1] + g1[1]], axis=1)
    h1 = jax.nn.relu(dinv * r + b1_ref[...])
    t = jnp.dot(h1, w2_ref[...], preferred_element_type=_f32,
                precision=lax.Precision.HIGHEST)
    _split(t, dinv, g2_ref)


_tc2 = pl.pallas_call(
    _tc2_body,
    grid=(GRID,),
    in_specs=[
        pl.BlockSpec((2, BR, DH), lambda i: (0, i, 0)),
        pl.BlockSpec((2, BR, DH), lambda i: (0, i, 0)),
        pl.BlockSpec((BR, 1), lambda i: (i, 0)),
        pl.BlockSpec((1, D), lambda i: (0, 0)),
        pl.BlockSpec((D, D), lambda i: (0, 0)),
    ],
    out_specs=pl.BlockSpec((2, BR, DH), lambda i: (0, i, 0)),
    out_shape=jax.ShapeDtypeStruct((NC, P, DH), _f32),
)


def _tc3_body(rp_ref, g2_ref, dinv_ref, b2_ref, sp_ref, w3_ref, b3_ref,
              out_ref, acc_ref):
    i = pl.program_id(0)
    rp = rp_ref[...]
    g2 = g2_ref[...]
    dinv = dinv_ref[...]
    r = jnp.concatenate([rp[0] + g2[0], rp[1] + g2[1]], axis=1)
    h2 = jax.nn.relu(dinv * r + b2_ref[...])
    sp = sp_ref[...]
    w = dinv * (sp[0] + sp[1] + dinv)
    rows = i * BR + lax.broadcasted_iota(jnp.int32, (BR, 1), 0)
    w = jnp.where(rows < N, w, 0.0)
    contrib = jnp.sum(w * h2, axis=0, keepdims=True)

    @pl.when(i == 0)
    def _():
        acc_ref[...] = contrib

    @pl.when(i > 0)
    def _():
        acc_ref[...] = acc_ref[...] + contrib

    @pl.when(i == GRID - 1)
    def _():
        u = acc_ref[...] * (1.0 / N)
        out_ref[...] = jnp.dot(u, w3_ref[...], preferred_element_type=_f32,
                               precision=lax.Precision.HIGHEST) + b3_ref[...]


_tc3 = pl.pallas_call(
    _tc3_body,
    grid=(GRID,),
    in_specs=[
        pl.BlockSpec((2, BR, DH), lambda i: (0, i, 0)),
        pl.BlockSpec((2, BR, DH), lambda i: (0, i, 0)),
        pl.BlockSpec((BR, 1), lambda i: (i, 0)),
        pl.BlockSpec((1, D), lambda i: (0, 0)),
        pl.BlockSpec((2, BR, 1), lambda i: (0, i, 0)),
        pl.BlockSpec((D, 64), lambda i: (0, 0)),
        pl.BlockSpec((1, 64), lambda i: (0, 0)),
    ],
    out_specs=pl.BlockSpec((1, 64), lambda i: (0, 0)),
    out_shape=jax.ShapeDtypeStruct((1, 64), _f32),
    scratch_shapes=[pltpu.VMEM((1, D), _f32)],
)


def kernel(x, edge_index, W1, b1, W2, b2, W3, b3):
    src32 = edge_index[0].astype(jnp.int32)
    dst32 = edge_index[1].astype(jnp.int32)
    src = src32.reshape(NS, NB, K)
    dst = dst32.reshape(NS, NB, K)
    dst_deg = dst32.reshape(NC, NS, NBD, K)
    xp = jnp.pad(x, ((0, P - N), (0, 0)))

    indeg_p = _deg_kernel(dst_deg)
    g1, dinv = _tc1(xp, W1, indeg_p.reshape(NC, P, 1))
    dinv_flat = dinv.reshape(P)
    r1, s_raw = _edge_kernel_s(g1, src, dst, dinv_flat)
    g2 = _tc2(r1, g1, dinv, b1.reshape(1, D), W2)
    (r2,) = _edge_kernel(g2, src, dst, dinv_flat)
    out = _tc3(r2, g2, dinv, b2.reshape(1, D), s_raw.reshape(NC, P, 1),
               W3, b3.reshape(1, 64))
    return out


# speedup vs baseline: 31.2652x; 1.0324x over previous
import functools

import jax
import jax.numpy as jnp
from jax import lax
from jax.experimental import pallas as pl
from jax.experimental.pallas import tpu as pltpu
from jax.experimental.pallas import tpu_sc as plsc

N = 10000
P = 10240
E = 320000
D = 128
DH = 64
NC, NS = 2, 16
K = 125
NB = E // (NS * K)
NBD = E // (NC * NS * K)
RPT = P // NS

_mesh = plsc.VectorSubcoreMesh(core_axis_name="c", subcore_axis_name="s")

_f32 = jnp.float32


def _zero_fill(buf, n_rows, width):
    def body(i, _):
        for j in range(width // 16):
            buf[i, pl.ds(j * 16, 16)] = jnp.zeros((16,), _f32)
        return 0
    lax.fori_loop(0, n_rows, body, 0, unroll=2)


@functools.partial(
    pl.kernel,
    out_type=jax.ShapeDtypeStruct((NC, P), _f32),
    mesh=_mesh,
    scratch_types=[
        pltpu.VMEM((NBD, K), jnp.int32),
        pltpu.VMEM((1, 128), _f32),
        pltpu.VMEM((128, 64), _f32),
        pltpu.VMEM_SHARED((P,), _f32),
        pltpu.SemaphoreType.DMA,
    ],
)
def _deg_kernel(dst_hbm, out_hbm, idx_v, ones_v, zer_v, acc_sh, sem):
    c = lax.axis_index("c")
    s = lax.axis_index("s")
    pltpu.sync_copy(dst_hbm.at[s, pl.ds(c * NBD, NBD)], idx_v)
    for j in range(8):
        ones_v[0, pl.ds(j * 16, 16)] = jnp.full((16,), 1.0, _f32)
    _zero_fill(zer_v, 10, 64)
    for k in range(10):
        pltpu.sync_copy(zer_v.at[0], acc_sh.at[pl.ds(s * RPT + k * 64, 64)])
    plsc.subcore_barrier()

    def body(j, _):
        pltpu.async_copy(ones_v.at[0, pl.ds(0, K)], acc_sh.at[idx_v.at[j]],
                         sem, add=True)
        return 0
    lax.fori_loop(0, NBD, body, 0)

    def drain(j, _):
        pltpu.make_async_copy(ones_v.at[0, pl.ds(0, K)],
                              acc_sh.at[idx_v.at[0]], sem).wait()
        return 0
    lax.fori_loop(0, NBD, drain, 0)
    plsc.subcore_barrier()
    pltpu.sync_copy(acc_sh.at[pl.ds(s * RPT, RPT)],
                    out_hbm.at[c, pl.ds(s * RPT, RPT)])


NBUF = 4
NROUNDS = NB // NBUF
NBS = NBD


def _make_edge_kernel(with_s):
    out_type = [jax.ShapeDtypeStruct((NC, P, DH), _f32)]
    scratch = [
        pltpu.VMEM((NB, K), jnp.int32),
        pltpu.VMEM((NB, K), jnp.int32),
        pltpu.VMEM((NBUF, K, DH), _f32),
        pltpu.VMEM((64, 64), _f32),
        pltpu.VMEM_SHARED((P, DH), _f32),
        pltpu.SemaphoreType.DMA((NBUF,)),
        pltpu.SemaphoreType.DMA((NBUF,)),
    ]
    if with_s:
        out_type.append(jax.ShapeDtypeStruct((NC, P), _f32))
        scratch += [
            pltpu.VMEM((NBS, K), _f32),
            pltpu.VMEM_SHARED((P,), _f32),
            pltpu.SemaphoreType.DMA,
            pltpu.SemaphoreType.DMA,
        ]

    def body(g_hbm, src_hbm, dst_hbm, dinv_hbm, *refs):
        if with_s:
            (r_out, s_out, src_v, dst_v, rows_v, zer_v, acc_sh, gsem, ssem,
             vals_v, s_sh, sgsem, sssem) = refs
        else:
            (r_out, src_v, dst_v, rows_v, zer_v, acc_sh, gsem,
             ssem) = refs
        c = lax.axis_index("c")
        s = lax.axis_index("s")
        pltpu.sync_copy(src_hbm.at[s], src_v)
        pltpu.sync_copy(dst_hbm.at[s], dst_v)

        if with_s:
            def sgather(j, _):
                pltpu.async_copy(dinv_hbm.at[dst_v.at[c * NBS + j]],
                                 vals_v.at[j], sgsem)
                return 0
            lax.fori_loop(0, NBS, sgather, 0)

        _zero_fill(zer_v, 64, 64)
        for k in range(10):
            pltpu.sync_copy(zer_v, acc_sh.at[pl.ds(s * RPT + k * 64, 64)])
        if with_s:
            for k in range(10):
                pltpu.sync_copy(zer_v.at[0],
                                s_sh.at[pl.ds(s * RPT + k * 64, 64)])
        plsc.subcore_barrier()

        for u in range(NBUF):
            pltpu.async_copy(g_hbm.at[c].at[src_v.at[u]], rows_v.at[u],
                             gsem.at[u])

        def loop(i, _):
            for u in range(NBUF):
                j = i * NBUF + u
                pltpu.make_async_copy(g_hbm.at[c].at[src_v.at[j]],
                                      rows_v.at[u], gsem.at[u]).wait()
                pltpu.async_copy(rows_v.at[u], acc_sh.at[dst_v.at[j]],
                                 ssem.at[u], add=True)
            for u in range(NBUF):
                j = i * NBUF + u
                pltpu.make_async_copy(rows_v.at[u], acc_sh.at[dst_v.at[j]],
                                      ssem.at[u]).wait()

                @pl.when(i < NROUNDS - 1)
                def _():
                    jn = (i + 1) * NBUF + u
                    pltpu.async_copy(g_hbm.at[c].at[src_v.at[jn]],
                                     rows_v.at[u], gsem.at[u])
            if with_s:
                @pl.when(i == 4)
                def _():
                    def sdrain_g(j, _):
                        pltpu.make_async_copy(dinv_hbm.at[dst_v.at[c * NBS]],
                                              vals_v.at[0], sgsem).wait()
                        return 0
                    lax.fori_loop(0, NBS, sdrain_g, 0)

                    def sfire(j, _):
                        pltpu.async_copy(vals_v.at[j],
                                         s_sh.at[src_v.at[c * NBS + j]],
                                         sssem, add=True)
                        return 0
                    lax.fori_loop(0, NBS, sfire, 0)
            return 0
        lax.fori_loop(0, NROUNDS, loop, 0)

        if with_s:
            def sdrain_s(j, _):
                pltpu.make_async_copy(vals_v.at[0], s_sh.at[src_v.at[c * NBS]],
                                      sssem).wait()
                return 0
            lax.fori_loop(0, NBS, sdrain_s, 0)
        plsc.subcore_barrier()
        for k in range(5):
            pltpu.sync_copy(acc_sh.at[pl.ds(s * RPT + k * 128, 128)],
                            r_out.at[c, pl.ds(s * RPT + k * 128, 128)])
        if with_s:
            pltpu.sync_copy(s_sh.at[pl.ds(s * RPT, RPT)],
                            s_out.at[c, pl.ds(s * RPT, RPT)])

    return pl.kernel(body, out_type=out_type, mesh=_mesh,
                     scratch_types=scratch,
                     compiler_params=pltpu.CompilerParams(
                         use_tc_tiling_on_sc=False))


_edge_kernel_s = _make_edge_kernel(True)
_edge_kernel = _make_edge_kernel(False)


BR = 1000
GRID = N // BR


def _split(t, dinv, out_ref):
    out_ref[0] = t[:, :DH] * dinv
    out_ref[1] = t[:, DH:] * dinv


def _tc1_body(x_ref, w1_ref, ip_ref, g1_ref, dinv_ref):
    ip = ip_ref[...]
    deg = 1.0 + ip[0] + ip[1]
    dinv = lax.rsqrt(deg)
    dinv_ref[...] = dinv
    t = jnp.dot(x_ref[...], w1_ref[...], preferred_element_type=_f32,
                precision=lax.Precision.HIGHEST)
    _split(t, dinv, g1_ref)


_tc1 = pl.pallas_call(
    _tc1_body,
    grid=(GRID,),
    in_specs=[
        pl.BlockSpec((BR, D), lambda i: (i, 0)),
        pl.BlockSpec((D, D), lambda i: (0, 0)),
        pl.BlockSpec((2, BR, 1), lambda i: (0, i, 0)),
    ],
    out_specs=[
        pl.BlockSpec((2, BR, DH), lambda i: (0, i, 0)),
        pl.BlockSpec((BR, 1), lambda i: (i, 0)),
    ],
    out_shape=[
        jax.ShapeDtypeStruct((NC, N, DH), _f32),
        jax.ShapeDtypeStruct((N, 1), _f32),
    ],
)


def _tc2_body(rp_ref, g1_ref, dinv_ref, b1_ref, w2_ref, g2_ref):
    rp = rp_ref[...]
    g1 = g1_ref[...]
    dinv = dinv_ref[...]
    r = jnp.concatenate([rp[0] + g1[0], rp[1] + g1[1]], axis=1)
    h1 = jax.nn.relu(dinv * r + b1_ref[...])
    t = jnp.dot(h1, w2_ref[...], preferred_element_type=_f32,
                precision=lax.Precision.HIGHEST)
    _split(t, dinv, g2_ref)


_tc2 = pl.pallas_call(
    _tc2_body,
    grid=(GRID,),
    in_specs=[
        pl.BlockSpec((2, BR, DH), lambda i: (0, i, 0)),
        pl.BlockSpec((2, BR, DH), lambda i: (0, i, 0)),
        pl.BlockSpec((BR, 1), lambda i: (i, 0)),
        pl.BlockSpec((1, D), lambda i: (0, 0)),
        pl.BlockSpec((D, D), lambda i: (0, 0)),
    ],
    out_specs=pl.BlockSpec((2, BR, DH), lambda i: (0, i, 0)),
    out_shape=jax.ShapeDtypeStruct((NC, N, DH), _f32),
)


def _tc3_body(rp_ref, g2_ref, dinv_ref, b2_ref, sp_ref, w3_ref, b3_ref,
              out_ref, acc_ref):
    i = pl.program_id(0)
    rp = rp_ref[...]
    g2 = g2_ref[...]
    dinv = dinv_ref[...]
    r = jnp.concatenate([rp[0] + g2[0], rp[1] + g2[1]], axis=1)
    h2 = jax.nn.relu(dinv * r + b2_ref[...])
    sp = sp_ref[...]
    w = dinv * (sp[0] + sp[1] + dinv)
    contrib = jnp.sum(w * h2, axis=0, keepdims=True)

    @pl.when(i == 0)
    def _():
        acc_ref[...] = contrib

    @pl.when(i > 0)
    def _():
        acc_ref[...] = acc_ref[...] + contrib

    @pl.when(i == GRID - 1)
    def _():
        u = acc_ref[...] * (1.0 / N)
        out_ref[...] = jnp.dot(u, w3_ref[...], preferred_element_type=_f32,
                               precision=lax.Precision.HIGHEST) + b3_ref[...]


_tc3 = pl.pallas_call(
    _tc3_body,
    grid=(GRID,),
    in_specs=[
        pl.BlockSpec((2, BR, DH), lambda i: (0, i, 0)),
        pl.BlockSpec((2, BR, DH), lambda i: (0, i, 0)),
        pl.BlockSpec((BR, 1), lambda i: (i, 0)),
        pl.BlockSpec((1, D), lambda i: (0, 0)),
        pl.BlockSpec((2, BR, 1), lambda i: (0, i, 0)),
        pl.BlockSpec((D, 64), lambda i: (0, 0)),
        pl.BlockSpec((1, 64), lambda i: (0, 0)),
    ],
    out_specs=pl.BlockSpec((1, 64), lambda i: (0, 0)),
    out_shape=jax.ShapeDtypeStruct((1, 64), _f32),
    scratch_shapes=[pltpu.VMEM((1, D), _f32)],
)


def kernel(x, edge_index, W1, b1, W2, b2, W3, b3):
    src32 = edge_index[0].astype(jnp.int32)
    dst32 = edge_index[1].astype(jnp.int32)
    src = src32.reshape(NS, NB, K)
    dst = dst32.reshape(NS, NB, K)
    indeg_p = _deg_kernel(dst)
    g1, dinv = _tc1(x, W1, indeg_p.reshape(NC, P, 1))
    dinv_flat = dinv.reshape(N)
    r1, s_raw = _edge_kernel_s(g1, src, dst, dinv_flat)
    g2 = _tc2(r1, g1, dinv, b1.reshape(1, D), W2)
    (r2,) = _edge_kernel(g2, src, dst, dinv_flat)
    out = _tc3(r2, g2, dinv, b2.reshape(1, D), s_raw.reshape(NC, P, 1),
               W3, b3.reshape(1, 64))
    return out


# speedup vs baseline: 31.2788x; 1.0004x over previous
import functools

import jax
import jax.numpy as jnp
from jax import lax
from jax.experimental import pallas as pl
from jax.experimental.pallas import tpu as pltpu
from jax.experimental.pallas import tpu_sc as plsc

N = 10000
P = 10240
E = 320000
D = 128
DH = 64
NC, NS = 2, 16
K = 125
NB = E // (NS * K)
NBD = E // (NC * NS * K)
RPT = P // NS

_mesh = plsc.VectorSubcoreMesh(core_axis_name="c", subcore_axis_name="s")

_f32 = jnp.float32


def _zero_fill(buf, n_rows, width):
    def body(i, _):
        for j in range(width // 16):
            buf[i, pl.ds(j * 16, 16)] = jnp.zeros((16,), _f32)
        return 0
    lax.fori_loop(0, n_rows, body, 0, unroll=2)


@functools.partial(
    pl.kernel,
    out_type=jax.ShapeDtypeStruct((NC, P), _f32),
    mesh=_mesh,
    scratch_types=[
        pltpu.VMEM((NBD, K), jnp.int32),
        pltpu.VMEM((1, 128), _f32),
        pltpu.VMEM((128, 64), _f32),
        pltpu.VMEM_SHARED((P,), _f32),
        pltpu.SemaphoreType.DMA,
    ],
)
def _deg_kernel(dst_hbm, out_hbm, idx_v, ones_v, zer_v, acc_sh, sem):
    c = lax.axis_index("c")
    s = lax.axis_index("s")
    pltpu.sync_copy(dst_hbm.at[s, pl.ds(c * NBD, NBD)], idx_v)
    for j in range(8):
        ones_v[0, pl.ds(j * 16, 16)] = jnp.full((16,), 1.0, _f32)
    _zero_fill(zer_v, 10, 64)
    for k in range(10):
        pltpu.sync_copy(zer_v.at[0], acc_sh.at[pl.ds(s * RPT + k * 64, 64)])
    plsc.subcore_barrier()

    def body(j, _):
        pltpu.async_copy(ones_v.at[0, pl.ds(0, K)], acc_sh.at[idx_v.at[j]],
                         sem, add=True)
        return 0
    lax.fori_loop(0, NBD, body, 0)

    def drain(j, _):
        pltpu.make_async_copy(ones_v.at[0, pl.ds(0, K)],
                              acc_sh.at[idx_v.at[0]], sem).wait()
        return 0
    lax.fori_loop(0, NBD, drain, 0)
    plsc.subcore_barrier()
    pltpu.sync_copy(acc_sh.at[pl.ds(s * RPT, RPT)],
                    out_hbm.at[c, pl.ds(s * RPT, RPT)])


NBS = NBD


def _make_edge_kernel(with_s, nbuf):
    nrounds = NB // nbuf
    out_type = [jax.ShapeDtypeStruct((NC, P, DH), _f32)]
    scratch = [
        pltpu.VMEM((NB, K), jnp.int32),
        pltpu.VMEM((NB, K), jnp.int32),
        pltpu.VMEM((nbuf, K, DH), _f32),
        pltpu.VMEM((64, 64), _f32),
        pltpu.VMEM_SHARED((P, DH), _f32),
        pltpu.SemaphoreType.DMA((nbuf,)),
        pltpu.SemaphoreType.DMA((nbuf,)),
    ]
    if with_s:
        out_type.append(jax.ShapeDtypeStruct((NC, P), _f32))
        scratch += [
            pltpu.VMEM((NBS, K), _f32),
            pltpu.VMEM_SHARED((P,), _f32),
            pltpu.SemaphoreType.DMA,
            pltpu.SemaphoreType.DMA,
        ]

    def body(g_hbm, src_hbm, dst_hbm, dinv_hbm, *refs):
        if with_s:
            (r_out, s_out, src_v, dst_v, rows_v, zer_v, acc_sh, gsem, ssem,
             vals_v, s_sh, sgsem, sssem) = refs
        else:
            (r_out, src_v, dst_v, rows_v, zer_v, acc_sh, gsem,
             ssem) = refs
        c = lax.axis_index("c")
        s = lax.axis_index("s")
        pltpu.sync_copy(src_hbm.at[s], src_v)
        pltpu.sync_copy(dst_hbm.at[s], dst_v)
        _zero_fill(zer_v, 64, 64)
        for k in range(10):
            pltpu.sync_copy(zer_v, acc_sh.at[pl.ds(s * RPT + k * 64, 64)])
        if with_s:
            for k in range(10):
                pltpu.sync_copy(zer_v.at[0],
                                s_sh.at[pl.ds(s * RPT + k * 64, 64)])
        plsc.subcore_barrier()

        for u in range(nbuf):
            pltpu.async_copy(g_hbm.at[c].at[src_v.at[u]], rows_v.at[u],
                             gsem.at[u])

        if with_s:
            def sgather(j, _):
                pltpu.async_copy(dinv_hbm.at[dst_v.at[c * NBS + j]],
                                 vals_v.at[j], sgsem)
                return 0
            lax.fori_loop(0, NBS, sgather, 0)

        def loop(i, _):
            for u in range(nbuf):
                j = i * nbuf + u
                pltpu.make_async_copy(g_hbm.at[c].at[src_v.at[j]],
                                      rows_v.at[u], gsem.at[u]).wait()
                pltpu.async_copy(rows_v.at[u], acc_sh.at[dst_v.at[j]],
                                 ssem.at[u], add=True)
            for u in range(nbuf):
                j = i * nbuf + u
                pltpu.make_async_copy(rows_v.at[u], acc_sh.at[dst_v.at[j]],
                                      ssem.at[u]).wait()

                @pl.when(i < nrounds - 1)
                def _():
                    jn = (i + 1) * nbuf + u
                    pltpu.async_copy(g_hbm.at[c].at[src_v.at[jn]],
                                     rows_v.at[u], gsem.at[u])
            if with_s:
                @pl.when(i == nrounds - 5)
                def _():
                    def sdrain_g(j, _):
                        pltpu.make_async_copy(dinv_hbm.at[dst_v.at[c * NBS]],
                                              vals_v.at[0], sgsem).wait()
                        return 0
                    lax.fori_loop(0, NBS, sdrain_g, 0)

                    def sfire(j, _):
                        pltpu.async_copy(vals_v.at[j],
                                         s_sh.at[src_v.at[c * NBS + j]],
                                         sssem, add=True)
                        return 0
                    lax.fori_loop(0, NBS, sfire, 0)
            return 0
        lax.fori_loop(0, nrounds, loop, 0)

        if with_s:
            def sdrain_s(j, _):
                pltpu.make_async_copy(vals_v.at[0], s_sh.at[src_v.at[c * NBS]],
                                      sssem).wait()
                return 0
            lax.fori_loop(0, NBS, sdrain_s, 0)
        plsc.subcore_barrier()
        for k in range(5):
            pltpu.sync_copy(acc_sh.at[pl.ds(s * RPT + k * 128, 128)],
                            r_out.at[c, pl.ds(s * RPT + k * 128, 128)])
        if with_s:
            pltpu.sync_copy(s_sh.at[pl.ds(s * RPT, RPT)],
                            s_out.at[c, pl.ds(s * RPT, RPT)])

    return pl.kernel(body, out_type=out_type, mesh=_mesh,
                     scratch_types=scratch,
                     compiler_params=pltpu.CompilerParams(
                         use_tc_tiling_on_sc=False))


_edge_kernel_s = _make_edge_kernel(True, 4)
_edge_kernel = _make_edge_kernel(False, 5)


BR = 1000
GRID = N // BR


def _split(t, dinv, out_ref):
    out_ref[0] = t[:, :DH] * dinv
    out_ref[1] = t[:, DH:] * dinv


def _tc1_body(x_ref, w1_ref, ip_ref, g1_ref, dinv_ref):
    ip = ip_ref[...]
    deg = 1.0 + ip[0] + ip[1]
    dinv = lax.rsqrt(deg)
    dinv_ref[...] = dinv
    t = jnp.dot(x_ref[...], w1_ref[...], preferred_element_type=_f32,
                precision=lax.Precision.HIGHEST)
    _split(t, dinv, g1_ref)


_tc1 = pl.pallas_call(
    _tc1_body,
    grid=(GRID,),
    in_specs=[
        pl.BlockSpec((BR, D), lambda i: (i, 0)),
        pl.BlockSpec((D, D), lambda i: (0, 0)),
        pl.BlockSpec((2, BR, 1), lambda i: (0, i, 0)),
    ],
    out_specs=[
        pl.BlockSpec((2, BR, DH), lambda i: (0, i, 0)),
        pl.BlockSpec((BR, 1), lambda i: (i, 0)),
    ],
    out_shape=[
        jax.ShapeDtypeStruct((NC, N, DH), _f32),
        jax.ShapeDtypeStruct((N, 1), _f32),
    ],
)


def _tc2_body(rp_ref, g1_ref, dinv_ref, b1_ref, w2_ref, g2_ref):
    rp = rp_ref[...]
    g1 = g1_ref[...]
    dinv = dinv_ref[...]
    r = jnp.concatenate([rp[0] + g1[0], rp[1] + g1[1]], axis=1)
    h1 = jax.nn.relu(dinv * r + b1_ref[...])
    t = jnp.dot(h1, w2_ref[...], preferred_element_type=_f32,
                precision=lax.Precision.HIGHEST)
    _split(t, dinv, g2_ref)


_tc2 = pl.pallas_call(
    _tc2_body,
    grid=(GRID,),
    in_specs=[
        pl.BlockSpec((2, BR, DH), lambda i: (0, i, 0)),
        pl.BlockSpec((2, BR, DH), lambda i: (0, i, 0)),
        pl.BlockSpec((BR, 1), lambda i: (i, 0)),
        pl.BlockSpec((1, D), lambda i: (0, 0)),
        pl.BlockSpec((D, D), lambda i: (0, 0)),
    ],
    out_specs=pl.BlockSpec((2, BR, DH), lambda i: (0, i, 0)),
    out_shape=jax.ShapeDtypeStruct((NC, N, DH), _f32),
)


def _tc3_body(rp_ref, g2_ref, dinv_ref, b2_ref, sp_ref, w3_ref, b3_ref,
              out_ref, acc_ref):
    i = pl.program_id(0)
    rp = rp_ref[...]
    g2 = g2_ref[...]
    dinv = dinv_ref[...]
    r = jnp.concatenate([rp[0] + g2[0], rp[1] + g2[1]], axis=1)
    h2 = jax.nn.relu(dinv * r + b2_ref[...])
    sp = sp_ref[...]
    w = dinv * (sp[0] + sp[1] + dinv)
    contrib = jnp.sum(w * h2, axis=0, keepdims=True)

    @pl.when(i == 0)
    def _():
        acc_ref[...] = contrib

    @pl.when(i > 0)
    def _():
        acc_ref[...] = acc_ref[...] + contrib

    @pl.when(i == GRID - 1)
    def _():
        u = acc_ref[...] * (1.0 / N)
        out_ref[...] = jnp.dot(u, w3_ref[...], preferred_element_type=_f32,
                               precision=lax.Precision.HIGHEST) + b3_ref[...]


_tc3 = pl.pallas_call(
    _tc3_body,
    grid=(GRID,),
    in_specs=[
        pl.BlockSpec((2, BR, DH), lambda i: (0, i, 0)),
        pl.BlockSpec((2, BR, DH), lambda i: (0, i, 0)),
        pl.BlockSpec((BR, 1), lambda i: (i, 0)),
        pl.BlockSpec((1, D), lambda i: (0, 0)),
        pl.BlockSpec((2, BR, 1), lambda i: (0, i, 0)),
        pl.BlockSpec((D, 64), lambda i: (0, 0)),
        pl.BlockSpec((1, 64), lambda i: (0, 0)),
    ],
    out_specs=pl.BlockSpec((1, 64), lambda i: (0, 0)),
    out_shape=jax.ShapeDtypeStruct((1, 64), _f32),
    scratch_shapes=[pltpu.VMEM((1, D), _f32)],
)


def kernel(x, edge_index, W1, b1, W2, b2, W3, b3):
    src32 = edge_index[0].astype(jnp.int32)
    dst32 = edge_index[1].astype(jnp.int32)
    src = src32.reshape(NS, NB, K)
    dst = dst32.reshape(NS, NB, K)
    indeg_p = _deg_kernel(dst)
    g1, dinv = _tc1(x, W1, indeg_p.reshape(NC, P, 1))
    dinv_flat = dinv.reshape(N)
    r1, s_raw = _edge_kernel_s(g1, src, dst, dinv_flat)
    g2 = _tc2(r1, g1, dinv, b1.reshape(1, D), W2)
    (r2,) = _edge_kernel(g2, src, dst, dinv_flat)
    out = _tc3(r2, g2, dinv, b2.reshape(1, D), s_raw.reshape(NC, P, 1),
               W3, b3.reshape(1, 64))
    return out


# speedup vs baseline: 31.7146x; 1.0139x over previous
import functools

import jax
import jax.numpy as jnp
from jax import lax
from jax.experimental import pallas as pl
from jax.experimental.pallas import tpu as pltpu
from jax.experimental.pallas import tpu_sc as plsc

N = 10000
P = 10240
E = 320000
D = 128
DH = 64
NC, NS = 2, 16
K = 125
NB = E // (NS * K)
NBD = E // (NC * NS * K)
RPT = P // NS

_mesh = plsc.VectorSubcoreMesh(core_axis_name="c", subcore_axis_name="s")

_f32 = jnp.float32


def _zero_fill(buf, n_rows, width):
    def body(i, _):
        for j in range(width // 16):
            buf[i, pl.ds(j * 16, 16)] = jnp.zeros((16,), _f32)
        return 0
    lax.fori_loop(0, n_rows, body, 0, unroll=2)


@functools.partial(
    pl.kernel,
    out_type=jax.ShapeDtypeStruct((NC, P), _f32),
    mesh=_mesh,
    scratch_types=[
        pltpu.VMEM((NBD, K), jnp.int32),
        pltpu.VMEM((1, 128), _f32),
        pltpu.VMEM((128, 64), _f32),
        pltpu.VMEM_SHARED((P,), _f32),
        pltpu.SemaphoreType.DMA,
    ],
)
def _deg_kernel(dst_hbm, out_hbm, idx_v, ones_v, zer_v, acc_sh, sem):
    c = lax.axis_index("c")
    s = lax.axis_index("s")
    pltpu.sync_copy(dst_hbm.at[s, pl.ds(c * NBD, NBD)], idx_v)
    for j in range(8):
        ones_v[0, pl.ds(j * 16, 16)] = jnp.full((16,), 1.0, _f32)
    _zero_fill(zer_v, 10, 64)
    for k in range(10):
        pltpu.sync_copy(zer_v.at[0], acc_sh.at[pl.ds(s * RPT + k * 64, 64)])
    plsc.subcore_barrier()

    def body(j, _):
        pltpu.async_copy(ones_v.at[0, pl.ds(0, K)], acc_sh.at[idx_v.at[j]],
                         sem, add=True)
        return 0
    lax.fori_loop(0, NBD, body, 0)

    def drain(j, _):
        pltpu.make_async_copy(ones_v.at[0, pl.ds(0, K)],
                              acc_sh.at[idx_v.at[0]], sem).wait()
        return 0
    lax.fori_loop(0, NBD, drain, 0)
    plsc.subcore_barrier()
    pltpu.sync_copy(acc_sh.at[pl.ds(s * RPT, RPT)],
                    out_hbm.at[c, pl.ds(s * RPT, RPT)])


NBS = NBD


def _make_edge_kernel(with_s, nbuf):
    nrounds = NB // nbuf
    out_type = [jax.ShapeDtypeStruct((NC, P, DH), _f32)]
    scratch = [
        pltpu.VMEM((NB, K), jnp.int32),
        pltpu.VMEM((NB, K), jnp.int32),
        pltpu.VMEM((nbuf, K, DH), _f32),
        pltpu.VMEM((64, 64), _f32),
        pltpu.VMEM_SHARED((P, DH), _f32),
        pltpu.SemaphoreType.DMA((nbuf,)),
        pltpu.SemaphoreType.DMA((nbuf,)),
    ]
    if with_s:
        out_type.append(jax.ShapeDtypeStruct((NC, P), _f32))
        scratch += [
            pltpu.VMEM((NBS, K), _f32),
            pltpu.VMEM_SHARED((P,), _f32),
            pltpu.SemaphoreType.DMA,
            pltpu.SemaphoreType.DMA,
        ]

    def body(g_hbm, src_hbm, dst_hbm, dinv_hbm, *refs):
        if with_s:
            (r_out, s_out, src_v, dst_v, rows_v, zer_v, acc_sh, gsem, ssem,
             vals_v, s_sh, sgsem, sssem) = refs
        else:
            (r_out, src_v, dst_v, rows_v, zer_v, acc_sh, gsem,
             ssem) = refs
        c = lax.axis_index("c")
        s = lax.axis_index("s")
        pltpu.sync_copy(src_hbm.at[s], src_v)
        pltpu.sync_copy(dst_hbm.at[s], dst_v)
        _zero_fill(zer_v, 64, 64)
        for k in range(10):
            pltpu.sync_copy(zer_v, acc_sh.at[pl.ds(s * RPT + k * 64, 64)])
        if with_s:
            for k in range(10):
                pltpu.sync_copy(zer_v.at[0],
                                s_sh.at[pl.ds(s * RPT + k * 64, 64)])
        plsc.subcore_barrier()

        for u in range(nbuf):
            pltpu.async_copy(g_hbm.at[c].at[src_v.at[u]], rows_v.at[u],
                             gsem.at[u])

        if with_s:
            def sgather(j, _):
                pltpu.async_copy(dinv_hbm.at[dst_v.at[c * NBS + j]],
                                 vals_v.at[j], sgsem)
                return 0
            lax.fori_loop(0, NBS, sgather, 0)

        def loop(i, _):
            for u in range(nbuf):
                j = i * nbuf + u
                pltpu.make_async_copy(g_hbm.at[c].at[src_v.at[j]],
                                      rows_v.at[u], gsem.at[u]).wait()
                pltpu.async_copy(rows_v.at[u], acc_sh.at[dst_v.at[j]],
                                 ssem.at[u], add=True)
            for u in range(nbuf):
                j = i * nbuf + u
                pltpu.make_async_copy(rows_v.at[u], acc_sh.at[dst_v.at[j]],
                                      ssem.at[u]).wait()

                @pl.when(i < nrounds - 1)
                def _():
                    jn = (i + 1) * nbuf + u
                    pltpu.async_copy(g_hbm.at[c].at[src_v.at[jn]],
                                     rows_v.at[u], gsem.at[u])
            if with_s:
                @pl.when(i == nrounds - 5)
                def _():
                    def sdrain_g(j, _):
                        pltpu.make_async_copy(dinv_hbm.at[dst_v.at[c * NBS]],
                                              vals_v.at[0], sgsem).wait()
                        return 0
                    lax.fori_loop(0, NBS, sdrain_g, 0)

                    def sfire(j, _):
                        pltpu.async_copy(vals_v.at[j],
                                         s_sh.at[src_v.at[c * NBS + j]],
                                         sssem, add=True)
                        return 0
                    lax.fori_loop(0, NBS, sfire, 0)
            return 0
        lax.fori_loop(0, nrounds, loop, 0)

        if with_s:
            def sdrain_s(j, _):
                pltpu.make_async_copy(vals_v.at[0], s_sh.at[src_v.at[c * NBS]],
                                      sssem).wait()
                return 0
            lax.fori_loop(0, NBS, sdrain_s, 0)
        plsc.subcore_barrier()
        for k in range(5):
            pltpu.sync_copy(acc_sh.at[pl.ds(s * RPT + k * 128, 128)],
                            r_out.at[c, pl.ds(s * RPT + k * 128, 128)])
        if with_s:
            pltpu.sync_copy(s_sh.at[pl.ds(s * RPT, RPT)],
                            s_out.at[c, pl.ds(s * RPT, RPT)])

    return pl.kernel(body, out_type=out_type, mesh=_mesh,
                     scratch_types=scratch,
                     compiler_params=pltpu.CompilerParams(
                         use_tc_tiling_on_sc=False))


_edge_kernel_s = _make_edge_kernel(True, 4)
_edge_kernel = _make_edge_kernel(False, 5)


BR = 1000
GRID = N // BR


def _split(t, dinv, out_ref):
    out_ref[0] = t[:, :DH] * dinv
    out_ref[1] = t[:, DH:] * dinv


def _tc1_body(x_ref, w1_ref, ip_ref, g1_ref, dinv_ref):
    ip = ip_ref[...]
    deg = 1.0 + ip[0] + ip[1]
    dinv = lax.rsqrt(deg)
    dinv_ref[...] = dinv
    t = jnp.dot(x_ref[...], w1_ref[...], preferred_element_type=_f32,
                precision=lax.Precision.DEFAULT)
    _split(t, dinv, g1_ref)


_tc1 = pl.pallas_call(
    _tc1_body,
    grid=(GRID,),
    in_specs=[
        pl.BlockSpec((BR, D), lambda i: (i, 0)),
        pl.BlockSpec((D, D), lambda i: (0, 0)),
        pl.BlockSpec((2, BR, 1), lambda i: (0, i, 0)),
    ],
    out_specs=[
        pl.BlockSpec((2, BR, DH), lambda i: (0, i, 0)),
        pl.BlockSpec((BR, 1), lambda i: (i, 0)),
    ],
    out_shape=[
        jax.ShapeDtypeStruct((NC, N, DH), _f32),
        jax.ShapeDtypeStruct((N, 1), _f32),
    ],
)


def _tc2_body(rp_ref, g1_ref, dinv_ref, b1_ref, w2_ref, g2_ref):
    rp = rp_ref[...]
    g1 = g1_ref[...]
    dinv = dinv_ref[...]
    r = jnp.concatenate([rp[0] + g1[0], rp[1] + g1[1]], axis=1)
    h1 = jax.nn.relu(dinv * r + b1_ref[...])
    t = jnp.dot(h1, w2_ref[...], preferred_element_type=_f32,
                precision=lax.Precision.DEFAULT)
    _split(t, dinv, g2_ref)


_tc2 = pl.pallas_call(
    _tc2_body,
    grid=(GRID,),
    in_specs=[
        pl.BlockSpec((2, BR, DH), lambda i: (0, i, 0)),
        pl.BlockSpec((2, BR, DH), lambda i: (0, i, 0)),
        pl.BlockSpec((BR, 1), lambda i: (i, 0)),
        pl.BlockSpec((1, D), lambda i: (0, 0)),
        pl.BlockSpec((D, D), lambda i: (0, 0)),
    ],
    out_specs=pl.BlockSpec((2, BR, DH), lambda i: (0, i, 0)),
    out_shape=jax.ShapeDtypeStruct((NC, N, DH), _f32),
)


def _tc3_body(rp_ref, g2_ref, dinv_ref, b2_ref, sp_ref, w3_ref, b3_ref,
              out_ref, acc_ref):
    i = pl.program_id(0)
    rp = rp_ref[...]
    g2 = g2_ref[...]
    dinv = dinv_ref[...]
    r = jnp.concatenate([rp[0] + g2[0], rp[1] + g2[1]], axis=1)
    h2 = jax.nn.relu(dinv * r + b2_ref[...])
    sp = sp_ref[...]
    w = dinv * (sp[0] + sp[1] + dinv)
    contrib = jnp.sum(w * h2, axis=0, keepdims=True)

    @pl.when(i == 0)
    def _():
        acc_ref[...] = contrib

    @pl.when(i > 0)
    def _():
        acc_ref[...] = acc_ref[...] + contrib

    @pl.when(i == GRID - 1)
    def _():
        u = acc_ref[...] * (1.0 / N)
        out_ref[...] = jnp.dot(u, w3_ref[...], preferred_element_type=_f32,
                               precision=lax.Precision.DEFAULT) + b3_ref[...]


_tc3 = pl.pallas_call(
    _tc3_body,
    grid=(GRID,),
    in_specs=[
        pl.BlockSpec((2, BR, DH), lambda i: (0, i, 0)),
        pl.BlockSpec((2, BR, DH), lambda i: (0, i, 0)),
        pl.BlockSpec((BR, 1), lambda i: (i, 0)),
        pl.BlockSpec((1, D), lambda i: (0, 0)),
        pl.BlockSpec((2, BR, 1), lambda i: (0, i, 0)),
        pl.BlockSpec((D, 64), lambda i: (0, 0)),
        pl.BlockSpec((1, 64), lambda i: (0, 0)),
    ],
    out_specs=pl.BlockSpec((1, 64), lambda i: (0, 0)),
    out_shape=jax.ShapeDtypeStruct((1, 64), _f32),
    scratch_shapes=[pltpu.VMEM((1, D), _f32)],
)


def kernel(x, edge_index, W1, b1, W2, b2, W3, b3):
    src32 = edge_index[0].astype(jnp.int32)
    dst32 = edge_index[1].astype(jnp.int32)
    src = src32.reshape(NS, NB, K)
    dst = dst32.reshape(NS, NB, K)
    indeg_p = _deg_kernel(dst)
    g1, dinv = _tc1(x, W1, indeg_p.reshape(NC, P, 1))
    dinv_flat = dinv.reshape(N)
    r1, s_raw = _edge_kernel_s(g1, src, dst, dinv_flat)
    g2 = _tc2(r1, g1, dinv, b1.reshape(1, D), W2)
    (r2,) = _edge_kernel(g2, src, dst, dinv_flat)
    out = _tc3(r2, g2, dinv, b2.reshape(1, D), s_raw.reshape(NC, P, 1),
               W3, b3.reshape(1, 64))
    return out
